# baseline (speedup 1.0000x reference)
import functools

import jax
import jax.numpy as jnp
from jax import lax
from jax.experimental import pallas as pl
from jax.experimental.pallas import tpu as pltpu
from jax.experimental.pallas import tpu_sc as plsc

_CHUNK = 64
_NBUF = 10


def kernel(vocab_ids, table):
    bsz, hist = vocab_ids.shape
    _, d = table.shape
    n = bsz * hist

    info = plsc.get_sparse_core_info()
    nw = info.num_cores * info.num_subcores
    n_per_w = n // nw
    n_chunks = n_per_w // _CHUNK
    n_groups = n_chunks // _NBUF
    assert n_per_w * nw == n and n_chunks * _CHUNK == n_per_w
    assert n_groups * _NBUF == n_chunks

    idx = vocab_ids.astype(jnp.int32).reshape(nw, n_chunks, _CHUNK)

    mesh = plsc.VectorSubcoreMesh(core_axis_name="c", subcore_axis_name="s")

    @functools.partial(
        pl.kernel,
        out_type=jax.ShapeDtypeStruct((n, d), table.dtype),
        mesh=mesh,
        scratch_types=[
            pltpu.VMEM((n_chunks, _CHUNK), jnp.int32),
            pltpu.VMEM((_NBUF, _CHUNK, d), jnp.float32),
            pltpu.SemaphoreType.DMA((_NBUF,)),
            pltpu.SemaphoreType.DMA((_NBUF,)),
        ],
    )
    def emb_lookup(idx_hbm, table_hbm, out_hbm, idx_v, bufs, gsem, wsem):
        wid = lax.axis_index("s") * info.num_cores + lax.axis_index("c")
        row0 = wid * n_per_w
        pltpu.sync_copy(idx_hbm.at[wid], idx_v)

        def gather(chunk, b):
            return pltpu.make_async_copy(
                table_hbm.at[idx_v.at[chunk]], bufs.at[b], gsem.at[b])

        def write(chunk, b):
            return pltpu.make_async_copy(
                bufs.at[b], out_hbm.at[pl.ds(row0 + chunk * _CHUNK, _CHUNK)],
                wsem.at[b])

        for b in range(_NBUF):
            gather(b, b).start()

        @pl.loop(0, n_groups)
        def _(g):
            c0 = g * _NBUF
            for b in range(_NBUF):
                gather(c0 + b, b).wait()
                write(c0 + b, b).start()
            for b in range(_NBUF):
                write(c0 + b, b).wait()
                nxt = c0 + _NBUF + b

                @pl.when(nxt < n_chunks)
                def _():
                    gather(nxt, b).start()

    out = emb_lookup(idx, table)
    return out.reshape(bsz, hist, d)


# speedup vs baseline: 1.7895x; 1.7895x over previous
import functools

import jax
import jax.numpy as jnp
from jax import lax
from jax.experimental import pallas as pl
from jax.experimental.pallas import tpu as pltpu
from jax.experimental.pallas import tpu_sc as plsc

_NBUF = 8


def kernel(vocab_ids, table):
    bsz, hist = vocab_ids.shape
    _, d = table.shape

    info = plsc.get_sparse_core_info()
    nw = info.num_cores * info.num_subcores
    bpw = bsz // nw
    n_groups = bpw // _NBUF
    assert bpw * nw == bsz and n_groups * _NBUF == bpw
    assert hist <= 128

    idx = vocab_ids.astype(jnp.int32)

    mesh = plsc.VectorSubcoreMesh(core_axis_name="c", subcore_axis_name="s")

    @functools.partial(
        pl.kernel,
        out_type=jax.ShapeDtypeStruct((bsz, hist, d), table.dtype),
        mesh=mesh,
        scratch_types=[
            pltpu.VMEM((bpw, hist), jnp.int32),
            pltpu.VMEM((_NBUF, hist, d), jnp.float32),
            pltpu.SemaphoreType.DMA((_NBUF,)),
            pltpu.SemaphoreType.DMA((_NBUF,)),
        ],
        compiler_params=pltpu.CompilerParams(use_tc_tiling_on_sc=True),
    )
    def emb_lookup(idx_hbm, table_hbm, out_hbm, idx_v, bufs, gsem, wsem):
        wid = lax.axis_index("s") * info.num_cores + lax.axis_index("c")
        b0 = wid * bpw
        pltpu.sync_copy(idx_hbm.at[pl.ds(b0, bpw)], idx_v)

        def gather(j, b):
            return pltpu.make_async_copy(
                table_hbm.at[idx_v.at[j]], bufs.at[b], gsem.at[b])

        def write(j, b):
            return pltpu.make_async_copy(bufs.at[b], out_hbm.at[b0 + j],
                                         wsem.at[b])

        for b in range(_NBUF):
            gather(b, b).start()

        @pl.loop(0, n_groups)
        def _(g):
            c0 = g * _NBUF
            for b in range(_NBUF):
                gather(c0 + b, b).wait()
                write(c0 + b, b).start()
            for b in range(_NBUF):
                write(c0 + b, b).wait()
                nxt = c0 + _NBUF + b

                @pl.when(nxt < bpw)
                def _():
                    gather(nxt, b).start()

    return emb_lookup(idx, table)


# speedup vs baseline: 3.0800x; 1.7211x over previous
import functools

import jax
import jax.numpy as jnp
from jax import lax
from jax.experimental import pallas as pl
from jax.experimental.pallas import tpu as pltpu
from jax.experimental.pallas import tpu_sc as plsc

_CHUNK = 128
_NBUF = 5


def kernel(vocab_ids, table):
    bsz, hist = vocab_ids.shape
    _, d = table.shape
    n = bsz * hist

    info = plsc.get_sparse_core_info()
    nw = info.num_cores * info.num_subcores
    n_per_w = n // nw
    n_chunks = n_per_w // _CHUNK
    n_groups = n_chunks // _NBUF
    assert n_per_w * nw == n and n_chunks * _CHUNK == n_per_w
    assert n_groups * _NBUF == n_chunks

    idx = vocab_ids.astype(jnp.int32).T.reshape(-1)

    mesh = plsc.VectorSubcoreMesh(core_axis_name="c", subcore_axis_name="s")

    @functools.partial(
        pl.kernel,
        out_type=jax.ShapeDtypeStruct((n, d), table.dtype),
        mesh=mesh,
        scratch_types=[
            pltpu.VMEM((n_per_w,), jnp.int32),
            pltpu.VMEM((_NBUF, _CHUNK, d), jnp.float32),
            pltpu.SemaphoreType.DMA((_NBUF,)),
            pltpu.SemaphoreType.DMA((_NBUF,)),
        ],
        compiler_params=pltpu.CompilerParams(use_tc_tiling_on_sc=True),
    )
    def emb_lookup(idx_hbm, table_hbm, out_hbm, idx_v, bufs, gsem, wsem):
        wid = lax.axis_index("s") * info.num_cores + lax.axis_index("c")
        row0 = wid * n_per_w
        pltpu.sync_copy(idx_hbm.at[pl.ds(row0, n_per_w)], idx_v)

        def gather(c, b):
            return pltpu.make_async_copy(
                table_hbm.at[idx_v.at[pl.ds(c * _CHUNK, _CHUNK)]],
                bufs.at[b], gsem.at[b])

        def write(c, b):
            return pltpu.make_async_copy(
                bufs.at[b], out_hbm.at[pl.ds(row0 + c * _CHUNK, _CHUNK)],
                wsem.at[b])

        for b in range(_NBUF):
            gather(b, b).start()

        @pl.loop(0, n_groups)
        def _(g):
            c0 = g * _NBUF
            for b in range(_NBUF):
                gather(c0 + b, b).wait()
                write(c0 + b, b).start()
            for b in range(_NBUF):
                write(c0 + b, b).wait()
                nxt = c0 + _NBUF + b

                @pl.when(nxt < n_chunks)
                def _():
                    gather(nxt, b).start()

    out = emb_lookup(idx, table)
    return out.reshape(hist, bsz, d).transpose(1, 0, 2)


# speedup vs baseline: 3.1303x; 1.0163x over previous
import functools

import jax
import jax.numpy as jnp
from jax import lax
from jax.experimental import pallas as pl
from jax.experimental.pallas import tpu as pltpu
from jax.experimental.pallas import tpu_sc as plsc

_CHUNK = 64
_NBUF = 10


def kernel(vocab_ids, table):
    bsz, hist = vocab_ids.shape
    _, d = table.shape
    n = bsz * hist

    info = plsc.get_sparse_core_info()
    nw = info.num_cores * info.num_subcores
    n_per_w = n // nw
    n_chunks = n_per_w // _CHUNK
    n_groups = n_chunks // _NBUF
    assert n_per_w * nw == n and n_chunks * _CHUNK == n_per_w
    assert n_groups * _NBUF == n_chunks

    idx = vocab_ids.astype(jnp.int32).T.reshape(-1)

    mesh = plsc.VectorSubcoreMesh(core_axis_name="c", subcore_axis_name="s")

    @functools.partial(
        pl.kernel,
        out_type=jax.ShapeDtypeStruct((n, d), table.dtype),
        mesh=mesh,
        scratch_types=[
            pltpu.VMEM((n_per_w,), jnp.int32),
            pltpu.VMEM((_NBUF, _CHUNK, d), jnp.float32),
            pltpu.SemaphoreType.DMA((_NBUF,)),
            pltpu.SemaphoreType.DMA((_NBUF,)),
        ],
        compiler_params=pltpu.CompilerParams(use_tc_tiling_on_sc=True),
    )
    def emb_lookup(idx_hbm, table_hbm, out_hbm, idx_v, bufs, gsem, wsem):
        wid = lax.axis_index("s") * info.num_cores + lax.axis_index("c")
        row0 = wid * n_per_w
        pltpu.sync_copy(idx_hbm.at[pl.ds(row0, n_per_w)], idx_v)

        def gather(c, b):
            return pltpu.make_async_copy(
                table_hbm.at[idx_v.at[pl.ds(c * _CHUNK, _CHUNK)]],
                bufs.at[b], gsem.at[b])

        def write(c, b):
            return pltpu.make_async_copy(
                bufs.at[b], out_hbm.at[pl.ds(row0 + c * _CHUNK, _CHUNK)],
                wsem.at[b])

        for b in range(_NBUF):
            gather(b, b).start()

        @pl.loop(0, n_groups)
        def _(g):
            c0 = g * _NBUF
            for b in range(_NBUF):
                gather(c0 + b, b).wait()
                write(c0 + b, b).start()
            for b in range(_NBUF):
                write(c0 + b, b).wait()
                nxt = c0 + _NBUF + b

                @pl.when(nxt < n_chunks)
                def _():
                    gather(nxt, b).start()

    out = emb_lookup(idx, table)
    return out.reshape(hist, bsz, d).transpose(1, 0, 2)
